# baseline (speedup 1.0000x reference)
import functools

import jax
import jax.numpy as jnp
from jax import lax
from jax.experimental import pallas as pl
from jax.experimental.pallas import tpu as pltpu
from jax.experimental.pallas import tpu_sc as plsc

_B, _S, _D = 2, 2048, 64
_K_MAX = 64
_NT = 32
_QPT = _S // _NT
_NB = 1 << 15
_BM = _NB - 1
_NB2 = 1 << 14
_BM2 = _NB2 - 1
_NCHUNK = _S // 16


def _sig_block(q_ref, k_ref, w0_ref, w1_ref, qsig, ksig):
    w0 = w0_ref[...]
    w1 = w1_ref[...]
    lane = lax.broadcasted_iota(jnp.int32, (_D, 4), 0)
    grp = lax.broadcasted_iota(jnp.int32, (_D, 4), 1)
    sel = (lane // 16) == grp
    pows = jnp.where(sel, (1 << jnp.mod(lane, 16)).astype(jnp.float32), 0.0)

    def sigs(x):
        proj0 = lax.dot_general(x[:, :32], w0, (((1,), (0,)), ((), ())))
        proj1 = lax.dot_general(x[:, 32:], w1, (((1,), (0,)), ((), ())))
        bits = (x > 0).astype(jnp.float32)
        halves = lax.dot_general(bits, pows, (((1,), (0,)), ((), ())))
        m = jnp.concatenate([proj0, proj1, halves], axis=1)
        mt = jnp.transpose(m)
        h = jnp.mod(jnp.floor(mt[:8] / 4.0), 32.0).astype(jnp.int32)
        hv = mt[8:].astype(jnp.int32)
        c0 = hv[0:1] | (hv[1:2] << 16)
        c1 = hv[2:3] | (hv[3:4] << 16)
        return jnp.concatenate([c0, h[:4], c1, h[4:]], axis=0)

    for b in range(_B):
        qsig[b] = sigs(q_ref[b])
        ksig[b] = sigs(k_ref[b])


def _signatures(query_up, key_up, W0, W1):
    sig_sd = jax.ShapeDtypeStruct((_B, 10, _S), jnp.int32)
    return pl.pallas_call(
        _sig_block,
        out_shape=[sig_sd, sig_sd],
    )(query_up, key_up, W0, W1)


def _splat(i):
    return jnp.full((16,), i, jnp.int32)


def _sread(ref, *idx):
    return plsc.load_gather(ref, [_splat(i) for i in idx])[0]


def _sc_join(qsig, ksig, query_up, key_up):
    mesh = plsc.VectorSubcoreMesh(core_axis_name="c", subcore_axis_name="s")

    @functools.partial(
        pl.kernel,
        mesh=mesh,
        compiler_params=pltpu.CompilerParams(needs_layout_passes=False),
        out_type=jax.ShapeDtypeStruct((_B, _S, _K_MAX), jnp.int32),
        scratch_types=[
            pltpu.VMEM((_NB,), jnp.int32),
            pltpu.VMEM((_NB2,), jnp.int32),
            pltpu.VMEM((10, _S), jnp.int32),
            pltpu.VMEM((10, 2 * _QPT), jnp.int32),
            pltpu.VMEM((10, 2 * _QPT), jnp.int32),
            pltpu.VMEM((_QPT, _D), jnp.float32),
            pltpu.VMEM((1, _D), jnp.float32),
            pltpu.VMEM((_QPT, _K_MAX), jnp.float32),
            pltpu.VMEM((_QPT, _K_MAX), jnp.float32),
            pltpu.VMEM((_QPT, _K_MAX), jnp.int32),
            pltpu.VMEM((_QPT, _K_MAX), jnp.int32),
            pltpu.VMEM((_QPT,), jnp.int32),
            pltpu.SemaphoreType.DMA,
            pltpu.SemaphoreType.DMA,
            pltpu.SemaphoreType.DMA,
        ],
    )
    def join(qsig_h, ksig_h, q_h, k_h, out_h,
             pres, pres2, kv, qv0, qv1, qvec, krow,
             lsc0, lsc1, lidx0, lidx1, lcnt,
             semk, semq, semo):
        wid = lax.axis_index("s") * 2 + lax.axis_index("c")
        base = wid * _QPT
        abase = (wid // 2) * (2 * _QPT)
        qoff = (wid % 2) * _QPT
        zeros16 = jnp.zeros((16,), jnp.int32)
        lane0 = lax.broadcasted_iota(jnp.int32, (16,), 0) == 0

        qvs = (qv0, qv1)
        lscs = (lsc0, lsc1)
        lidxs = (lidx0, lidx1)

        def swrite(ref, val, *idx):
            plsc.store_scatter(ref, [_splat(i) for i in idx],
                               jnp.full((16,), val), mask=lane0)

        def insert(lsc, lidx, qi, kk, sc, cnt):
            def pcond(p):
                pm = jnp.minimum(p, _K_MAX - 1)
                sp = _sread(lsc, qi, pm)
                ip = _sread(lidx, qi, pm)
                return (p < cnt) & ((sp > sc) | ((sp == sc) & (ip < kk)))

            pos = lax.while_loop(pcond, lambda p: p + 1, jnp.int32(0))

            @pl.when(pos < _K_MAX)
            def _():
                end = jnp.minimum(cnt, _K_MAX - 1)

                def sbody(j):
                    swrite(lsc, _sread(lsc, qi, j - 1), qi, j)
                    swrite(lidx, _sread(lidx, qi, j - 1), qi, j)
                    return j - 1

                lax.while_loop(lambda j: j > pos, sbody, end)
                swrite(lsc, sc, qi, pos)
                swrite(lidx, kk, qi, pos)
                swrite(lcnt, jnp.minimum(cnt + 1, _K_MAX), qi)

        def scan_keys(b, g, qi, qcode_s):
            qv = qvs[b]
            lsc, lidx = lscs[b], lidxs[b]
            qsp = _splat(qcode_s)
            r0 = 5 * g
            qh = [_splat(_sread(qv, r0 + 1 + h, qoff + qi)) for h in range(4)]

            def process_chunk(ci):
                o = ci * 16
                code_eq = kv[r0, pl.ds(o, 16)] == qsp
                hm = ((kv[r0 + 1, pl.ds(o, 16)] == qh[0])
                      | (kv[r0 + 2, pl.ds(o, 16)] == qh[1])
                      | (kv[r0 + 3, pl.ds(o, 16)] == qh[2])
                      | (kv[r0 + 4, pl.ds(o, 16)] == qh[3]))
                cand = code_eq & hm

                @pl.when(jnp.max(jnp.where(cand, 1, 0)) > 0)
                def _():
                    def perk(j, _):
                        kk = o + j
                        ceq = _sread(kv, r0, kk) == qcode_s
                        hms = ((_sread(kv, r0 + 1, kk) == qh[0][0])
                               | (_sread(kv, r0 + 2, kk) == qh[1][0])
                               | (_sread(kv, r0 + 3, kk) == qh[2][0])
                               | (_sread(kv, r0 + 4, kk) == qh[3][0]))

                        @pl.when(ceq & hms)
                        def _():
                            cnt = _sread(lcnt, qi)
                            lane = lax.broadcasted_iota(jnp.int32, (16,), 0)
                            dup = jnp.int32(0)
                            for cc in range(_K_MAX // 16):
                                row = lidx[qi, pl.ds(cc * 16, 16)]
                                valid = (cc * 16 + lane) < _splat(cnt)
                                dup = dup | jnp.max(jnp.where(
                                    (row == kk) & valid, 1, 0))

                            @pl.when(dup == 0)
                            def _():
                                pltpu.sync_copy(
                                    k_h.at[b, pl.ds(kk, 1), :], krow)
                                sc = jnp.float32(0.0)
                                for r in range(_D // 16):
                                    sc = sc + jnp.sum(
                                        qvec[qi, pl.ds(r * 16, 16)]
                                        * krow[0, pl.ds(r * 16, 16)])
                                insert(lsc, lidx, qi, kk, sc, cnt)

                        return 0

                    lax.fori_loop(0, 16, perk, 0)

            def next_match(start):
                @plsc.parallel_loop(start, _NCHUNK, carry=jnp.int32(_NCHUNK),
                                    unroll=4)
                def first(ci, acc):
                    eq = kv[r0, pl.ds(ci * 16, 16)] == qsp
                    hit = jnp.max(jnp.where(eq, 1, 0))
                    return jnp.minimum(acc, jnp.where(hit > 0, ci, _NCHUNK))

                return first

            def wbody(start):
                f = next_match(start)

                @pl.when(f < _NCHUNK)
                def _():
                    process_chunk(f)

                return jnp.where(f < _NCHUNK, f + 1, jnp.int32(_NCHUNK))

            lax.while_loop(lambda s: s < _NCHUNK, wbody, jnp.int32(0))

        wk = pltpu.async_copy(ksig_h.at[0], kv, semk)
        wv = pltpu.async_copy(q_h.at[0, pl.ds(base, _QPT), :], qvec, semk)
        wq = [pltpu.async_copy(qsig_h.at[b, :, pl.ds(abase, 2 * _QPT)],
                               qvs[b], semq)
              for b in range(_B)]

        @plsc.parallel_loop(0, _NB, step=16, unroll=8)
        def _(i):
            pres[pl.ds(i, 16)] = zeros16

        @plsc.parallel_loop(0, _NB2, step=16, unroll=8)
        def _(i):
            pres2[pl.ds(i, 16)] = zeros16

        wq[0].wait()
        wq[1].wait()

        out_waits = []
        for b in range(_B):
            qv = qvs[b]
            lsc, lidx = lscs[b], lidxs[b]

            for c in range(_QPT // 16):
                lcnt[pl.ds(c * 16, 16)] = zeros16
            wk.wait()
            wv.wait()

            def per_group(g, _, b=b, kv=kv, qv=qv):
                r0 = 5 * g
                epoch = _splat(1 + 2 * b + g)

                @plsc.parallel_loop(0, _S, step=16, unroll=8)
                def _(ci):
                    ch = kv[r0, pl.ds(ci, 16)]
                    plsc.store_scatter(pres, [ch & _BM], epoch)
                    h2 = lax.shift_right_logical(ch, 18) & _BM2
                    plsc.store_scatter(pres2, [h2], epoch)

                def per_qc(qc, _):
                    qcodes = qv[r0, pl.ds(qoff + qc * 16, 16)]
                    hits = plsc.load_gather(pres, [qcodes & _BM])
                    h2 = lax.shift_right_logical(qcodes, 18) & _BM2
                    hits2 = plsc.load_gather(pres2, [h2])
                    both = (hits == epoch) & (hits2 == epoch)

                    @pl.when(jnp.max(jnp.where(both, 1, 0)) > 0)
                    def _():
                        def perq(l, _):
                            qi = qc * 16 + l
                            qcode_s = _sread(qv, r0, qoff + qi)
                            hit1 = _sread(pres, qcode_s & _BM) == epoch[0]
                            h2s = lax.shift_right_logical(
                                qcode_s, 18) & _BM2
                            hit2 = _sread(pres2, h2s) == epoch[0]

                            @pl.when(hit1 & hit2)
                            def _():
                                scan_keys(b, g, qi, qcode_s)

                            return 0

                        lax.fori_loop(0, 16, perq, 0)

                    return 0

                lax.fori_loop(0, _QPT // 16, per_qc, 0)
                return 0

            lax.fori_loop(0, 2, per_group, 0)

            if b == 0:
                wk = pltpu.async_copy(ksig_h.at[1], kv, semk)
                wv = pltpu.async_copy(
                    q_h.at[1, pl.ds(base, _QPT), :], qvec, semk)

            lane = lax.broadcasted_iota(jnp.int32, (16,), 0)

            @plsc.parallel_loop(0, _QPT, unroll=2)
            def _(r):
                cnt = _splat(_sread(lcnt, r))
                for c in range(_K_MAX // 16):
                    sv = lsc[r, pl.ds(c * 16, 16)]
                    iv = lidx[r, pl.ds(c * 16, 16)]
                    keep = ((c * 16 + lane) < cnt) & (
                        sv > jnp.float32(-1e8))
                    lidx[r, pl.ds(c * 16, 16)] = jnp.where(keep, iv, -1)
            out_waits.append(pltpu.async_copy(
                lidx, out_h.at[b, pl.ds(base, _QPT), :], semo))

        for w in out_waits:
            w.wait()

    return join(qsig, ksig, query_up, key_up)


def kernel(query_up, key_up, W0, W1, head_idx=0):
    qsig, ksig = _signatures(query_up, key_up, W0, W1)
    return _sc_join(qsig, ksig, query_up, key_up)


# speedup vs baseline: 1.0983x; 1.0983x over previous
import functools

import jax
import jax.numpy as jnp
from jax import lax
from jax.experimental import pallas as pl
from jax.experimental.pallas import tpu as pltpu
from jax.experimental.pallas import tpu_sc as plsc

_B, _S, _D = 2, 2048, 64
_K_MAX = 64
_NT = 32
_QPT = _S // _NT
_NB = 1 << 15
_BM = _NB - 1
_NB2 = 1 << 14
_BM2 = _NB2 - 1
_NCHUNK = _S // 16


def _sig_block(q_ref, k_ref, w0_ref, w1_ref, qsig, ksig):
    w0 = w0_ref[...]
    w1 = w1_ref[...]
    lane = lax.broadcasted_iota(jnp.int32, (_D, 4), 0)
    grp = lax.broadcasted_iota(jnp.int32, (_D, 4), 1)
    sel = (lane // 16) == grp
    pows = jnp.where(sel, (1 << jnp.mod(lane, 16)).astype(jnp.float32), 0.0)

    def sigs(x):
        proj0 = lax.dot_general(x[:, :32], w0, (((1,), (0,)), ((), ())))
        proj1 = lax.dot_general(x[:, 32:], w1, (((1,), (0,)), ((), ())))
        bits = (x > 0).astype(jnp.float32)
        halves = lax.dot_general(bits, pows, (((1,), (0,)), ((), ())))
        m = jnp.concatenate([proj0, proj1, halves], axis=1)
        mt = jnp.transpose(m)
        h = jnp.mod(jnp.floor(mt[:8] / 4.0), 32.0).astype(jnp.int32)
        hv = mt[8:].astype(jnp.int32)
        c0 = hv[0:1] | (hv[1:2] << 16)
        c1 = hv[2:3] | (hv[3:4] << 16)
        return jnp.concatenate([h, c0, c1], axis=0)

    for b in range(_B):
        qsig[b] = sigs(q_ref[b])
        ksig[b] = sigs(k_ref[b])


def _signatures(query_up, key_up, W0, W1):
    sig_sd = jax.ShapeDtypeStruct((_B, 10, _S), jnp.int32)
    return pl.pallas_call(
        _sig_block,
        out_shape=[sig_sd, sig_sd],
    )(query_up, key_up, W0, W1)


def _splat(i):
    return jnp.full((16,), i, jnp.int32)


def _sread(ref, *idx):
    return plsc.load_gather(ref, [_splat(i) for i in idx])[0]


def _sc_join(qsig, ksig, query_up, key_up):
    mesh = plsc.VectorSubcoreMesh(core_axis_name="c", subcore_axis_name="s")

    @functools.partial(
        pl.kernel,
        mesh=mesh,
        compiler_params=pltpu.CompilerParams(needs_layout_passes=False),
        out_type=jax.ShapeDtypeStruct((_B, _S, _K_MAX), jnp.int32),
        scratch_types=[
            pltpu.VMEM((_NB,), jnp.int32),
            pltpu.VMEM((_NB2,), jnp.int32),
            pltpu.VMEM((2, _S), jnp.int32),
            pltpu.VMEM((10, 2 * _QPT), jnp.int32),
            pltpu.VMEM((10, 2 * _QPT), jnp.int32),
            pltpu.VMEM((8, 128), jnp.int32),
            pltpu.VMEM((1, _D), jnp.float32),
            pltpu.VMEM((1, _D), jnp.float32),
            pltpu.VMEM((_QPT, _K_MAX), jnp.float32),
            pltpu.VMEM((_QPT, _K_MAX), jnp.float32),
            pltpu.VMEM((_QPT, _K_MAX), jnp.int32),
            pltpu.VMEM((_QPT, _K_MAX), jnp.int32),
            pltpu.VMEM((_QPT,), jnp.int32),
            pltpu.SemaphoreType.DMA,
            pltpu.SemaphoreType.DMA,
            pltpu.SemaphoreType.DMA,
        ],
    )
    def join(qsig_h, ksig_h, q_h, k_h, out_h,
             pres, pres2, kv, qv0, qv1, khwin, qrow, krow,
             lsc0, lsc1, lidx0, lidx1, lcnt,
             semk, semq, semo):
        wid = lax.axis_index("s") * 2 + lax.axis_index("c")
        base = wid * _QPT
        abase = (wid // 2) * (2 * _QPT)
        qoff = (wid % 2) * _QPT
        zeros16 = jnp.zeros((16,), jnp.int32)
        lane0 = lax.broadcasted_iota(jnp.int32, (16,), 0) == 0

        qvs = (qv0, qv1)
        lscs = (lsc0, lsc1)
        lidxs = (lidx0, lidx1)

        def swrite(ref, val, *idx):
            plsc.store_scatter(ref, [_splat(i) for i in idx],
                               jnp.full((16,), val), mask=lane0)

        def insert(lsc, lidx, qi, kk, sc, cnt):
            def pcond(p):
                pm = jnp.minimum(p, _K_MAX - 1)
                sp = _sread(lsc, qi, pm)
                ip = _sread(lidx, qi, pm)
                return (p < cnt) & ((sp > sc) | ((sp == sc) & (ip < kk)))

            pos = lax.while_loop(pcond, lambda p: p + 1, jnp.int32(0))

            @pl.when(pos < _K_MAX)
            def _():
                end = jnp.minimum(cnt, _K_MAX - 1)

                def sbody(j):
                    swrite(lsc, _sread(lsc, qi, j - 1), qi, j)
                    swrite(lidx, _sread(lidx, qi, j - 1), qi, j)
                    return j - 1

                lax.while_loop(lambda j: j > pos, sbody, end)
                swrite(lsc, sc, qi, pos)
                swrite(lidx, kk, qi, pos)
                swrite(lcnt, jnp.minimum(cnt + 1, _K_MAX), qi)

        def scan_keys(b, g, qi, qcode_s):
            qv = qvs[b]
            lsc, lidx = lscs[b], lidxs[b]
            qsp = _splat(qcode_s)
            qh = [_splat(_sread(qv, 4 * g + h, qoff + qi)) for h in range(4)]

            def process_chunk(ci):
                o = ci * 16
                ao = pl.multiple_of((ci // 8) * 128, 128)
                off = (ci % 8) * 16
                pltpu.sync_copy(
                    ksig_h.at[b, pl.ds(0, 8), pl.ds(ao, 128)], khwin)
                code_eq = kv[g, pl.ds(o, 16)] == qsp
                hm = ((khwin[4 * g, pl.ds(off, 16)] == qh[0])
                      | (khwin[4 * g + 1, pl.ds(off, 16)] == qh[1])
                      | (khwin[4 * g + 2, pl.ds(off, 16)] == qh[2])
                      | (khwin[4 * g + 3, pl.ds(off, 16)] == qh[3]))
                cand = code_eq & hm

                @pl.when(jnp.max(jnp.where(cand, 1, 0)) > 0)
                def _():
                    def perk(j, _):
                        kk = o + j
                        ceq = _sread(kv, g, kk) == qcode_s
                        hms = ((_sread(khwin, 4 * g, off + j) == qh[0][0])
                               | (_sread(khwin, 4 * g + 1, off + j)
                                  == qh[1][0])
                               | (_sread(khwin, 4 * g + 2, off + j)
                                  == qh[2][0])
                               | (_sread(khwin, 4 * g + 3, off + j)
                                  == qh[3][0]))

                        @pl.when(ceq & hms)
                        def _():
                            cnt = _sread(lcnt, qi)
                            lane = lax.broadcasted_iota(jnp.int32, (16,), 0)
                            dup = jnp.int32(0)
                            for cc in range(_K_MAX // 16):
                                row = lidx[qi, pl.ds(cc * 16, 16)]
                                valid = (cc * 16 + lane) < _splat(cnt)
                                dup = dup | jnp.max(jnp.where(
                                    (row == kk) & valid, 1, 0))

                            @pl.when(dup == 0)
                            def _():
                                pltpu.sync_copy(
                                    k_h.at[b, pl.ds(kk, 1), :], krow)
                                pltpu.sync_copy(
                                    q_h.at[b, pl.ds(base + qi, 1), :], qrow)
                                sc = jnp.float32(0.0)
                                for r in range(_D // 16):
                                    sc = sc + jnp.sum(
                                        qrow[0, pl.ds(r * 16, 16)]
                                        * krow[0, pl.ds(r * 16, 16)])
                                insert(lsc, lidx, qi, kk, sc, cnt)

                        return 0

                    lax.fori_loop(0, 16, perk, 0)

            def next_match(start):
                @plsc.parallel_loop(start, _NCHUNK, carry=jnp.int32(_NCHUNK),
                                    unroll=4)
                def first(ci, acc):
                    eq = kv[g, pl.ds(ci * 16, 16)] == qsp
                    hit = jnp.max(jnp.where(eq, 1, 0))
                    return jnp.minimum(acc, jnp.where(hit > 0, ci, _NCHUNK))

                return first

            def wbody(start):
                f = next_match(start)

                @pl.when(f < _NCHUNK)
                def _():
                    process_chunk(f)

                return jnp.where(f < _NCHUNK, f + 1, jnp.int32(_NCHUNK))

            lax.while_loop(lambda s: s < _NCHUNK, wbody, jnp.int32(0))

        wk = pltpu.async_copy(ksig_h.at[0, pl.ds(8, 2)], kv, semk)
        wq = [pltpu.async_copy(qsig_h.at[b, :, pl.ds(abase, 2 * _QPT)],
                               qvs[b], semq)
              for b in range(_B)]

        @plsc.parallel_loop(0, _NB, step=16, unroll=8)
        def _(i):
            pres[pl.ds(i, 16)] = zeros16

        @plsc.parallel_loop(0, _NB2, step=16, unroll=8)
        def _(i):
            pres2[pl.ds(i, 16)] = zeros16

        wq[0].wait()
        wq[1].wait()

        out_waits = []
        for b in range(_B):
            qv = qvs[b]
            lsc, lidx = lscs[b], lidxs[b]

            for c in range(_QPT // 16):
                lcnt[pl.ds(c * 16, 16)] = zeros16
            wk.wait()

            def per_group(g, _, b=b, kv=kv, qv=qv):
                epoch = _splat(1 + 2 * b + g)

                @plsc.parallel_loop(0, _S, step=16, unroll=8)
                def _(ci):
                    ch = kv[g, pl.ds(ci, 16)]
                    plsc.store_scatter(pres, [ch & _BM], epoch)
                    h2 = lax.shift_right_logical(ch, 18) & _BM2
                    plsc.store_scatter(pres2, [h2], epoch)

                def per_qc(qc, _):
                    qcodes = qv[8 + g, pl.ds(qoff + qc * 16, 16)]
                    hits = plsc.load_gather(pres, [qcodes & _BM])
                    h2 = lax.shift_right_logical(qcodes, 18) & _BM2
                    hits2 = plsc.load_gather(pres2, [h2])
                    both = (hits == epoch) & (hits2 == epoch)

                    @pl.when(jnp.max(jnp.where(both, 1, 0)) > 0)
                    def _():
                        def perq(l, _):
                            qi = qc * 16 + l
                            qcode_s = _sread(qv, 8 + g, qoff + qi)
                            hit1 = _sread(pres, qcode_s & _BM) == epoch[0]
                            h2s = lax.shift_right_logical(
                                qcode_s, 18) & _BM2
                            hit2 = _sread(pres2, h2s) == epoch[0]

                            @pl.when(hit1 & hit2)
                            def _():
                                scan_keys(b, g, qi, qcode_s)

                            return 0

                        lax.fori_loop(0, 16, perq, 0)

                    return 0

                lax.fori_loop(0, _QPT // 16, per_qc, 0)
                return 0

            lax.fori_loop(0, 2, per_group, 0)

            if b == 0:
                wk = pltpu.async_copy(ksig_h.at[1, pl.ds(8, 2)], kv, semk)

            lane = lax.broadcasted_iota(jnp.int32, (16,), 0)

            @plsc.parallel_loop(0, _QPT, unroll=2)
            def _(r):
                cnt = _splat(_sread(lcnt, r))
                for c in range(_K_MAX // 16):
                    sv = lsc[r, pl.ds(c * 16, 16)]
                    iv = lidx[r, pl.ds(c * 16, 16)]
                    keep = ((c * 16 + lane) < cnt) & (
                        sv > jnp.float32(-1e8))
                    lidx[r, pl.ds(c * 16, 16)] = jnp.where(keep, iv, -1)
            out_waits.append(pltpu.async_copy(
                lidx, out_h.at[b, pl.ds(base, _QPT), :], semo))

        for w in out_waits:
            w.wait()

    return join(qsig, ksig, query_up, key_up)


def kernel(query_up, key_up, W0, W1, head_idx=0):
    qsig, ksig = _signatures(query_up, key_up, W0, W1)
    return _sc_join(qsig, ksig, query_up, key_up)
